# baseline (speedup 1.0000x reference)
import functools

import jax
import jax.numpy as jnp
import numpy as np
from jax.experimental import pallas as pl
from jax.experimental.pallas import tpu as pltpu

_KS = 32
_PD = _KS // 2
_N = 512
_EPS = 1e-10
_CLAMP = 6.0

_dot = functools.partial(jnp.dot, preferred_element_type=jnp.float32)


def _box_count_matrix() -> np.ndarray:
    w = np.zeros((_N, _N), np.float32)
    for j in range(_N):
        for k in range(j, j + _KS):
            m = k - _PD
            if m < 0:
                m = -m
            elif m > _N - 1:
                m = 2 * (_N - 1) - m
            w[m, j] += 1.0
    return w


_W_NP = _box_count_matrix()

_SLAB_STARTS = (0, 64, 192, 256)
_WT_BLOCKS_NP = np.stack(
    [
        _W_NP.T[128 * b : 128 * b + 128, s : s + 256]
        for b, s in enumerate(_SLAB_STARTS)
    ]
)


_IPB = 12


_F8 = jnp.float8_e4m3fn


_GRP = 3


def _norm_kernel(x_ref, w_ref, wtb_ref, o_ref):
    w = w_ref[...]
    for g in range(_IPB // _GRP):
        imgs = range(g * _GRP, (g + 1) * _GRP)
        pieces = []
        for i in imgs:
            x = x_ref[i]
            pieces.append(x.astype(_F8))
            pieces.append((x * x).astype(_F8))
        h = jnp.concatenate(
            [_dot(p, w).astype(_F8) for p in pieces], axis=0
        )
        for b, s in enumerate(_SLAB_STARTS):
            rhs = jnp.concatenate(
                [h[k * _N + s : k * _N + s + 256] for k in range(2 * _GRP)], axis=1
            )
            vb = _dot(wtb_ref[b], rhs)
            for j, i in enumerate(imgs):
                mean = vb[:, 2 * j * _N : (2 * j + 1) * _N]
                meansq = vb[:, (2 * j + 1) * _N : (2 * j + 2) * _N]
                var = jnp.maximum(jnp.abs(meansq - mean * mean), 1e-20)
                xb = x_ref[i, 128 * b : 128 * (b + 1), :]
                z = (xb - mean) * jax.lax.rsqrt(var)
                o_ref[i, 128 * b : 128 * (b + 1), :] = jax.lax.clamp(
                    -_CLAMP, z, _CLAMP
                )


def kernel(x):
    b, c, h, wd = x.shape
    n_img = b * c
    xi = x.reshape(n_img, h, wd)
    w = jnp.asarray(_W_NP / 32.0, dtype=_F8)
    wtb = jnp.asarray(_WT_BLOCKS_NP / 32.0, dtype=_F8)
    out = pl.pallas_call(
        _norm_kernel,
        out_shape=jax.ShapeDtypeStruct((n_img, h, wd), x.dtype),
        grid=(n_img // _IPB,),
        in_specs=[
            pl.BlockSpec((_IPB, h, wd), lambda i: (i, 0, 0)),
            pl.BlockSpec((h, wd), lambda i: (0, 0)),
            pl.BlockSpec((4, 128, 256), lambda i: (0, 0, 0)),
        ],
        out_specs=pl.BlockSpec((_IPB, h, wd), lambda i: (i, 0, 0)),
        compiler_params=pltpu.CompilerParams(
            dimension_semantics=("arbitrary",),
            vmem_limit_bytes=56 * 1024 * 1024,
        ),
        name="local_norm2d",
    )(xi, w, wtb)
    return out.reshape(b, c, h, wd)


# speedup vs baseline: 1.0227x; 1.0227x over previous
import functools

import jax
import jax.numpy as jnp
import numpy as np
from jax.experimental import pallas as pl
from jax.experimental.pallas import tpu as pltpu

_KS = 32
_PD = _KS // 2
_N = 512
_CLAMP = 6.0

_dot = functools.partial(jnp.dot, preferred_element_type=jnp.float32)


def _box_count_matrix() -> np.ndarray:
    w = np.zeros((_N, _N), np.float32)
    for j in range(_N):
        for k in range(j, j + _KS):
            m = k - _PD
            if m < 0:
                m = -m
            elif m > _N - 1:
                m = 2 * (_N - 1) - m
            w[m, j] += 1.0
    return w


_W_NP = _box_count_matrix()

_SLAB_STARTS = (0, 64, 192, 256)
_WT_BLOCKS_NP = np.stack(
    [
        _W_NP.T[128 * b : 128 * b + 128, s : s + 256]
        for b, s in enumerate(_SLAB_STARTS)
    ]
)


_IPB = 12


_F8 = jnp.float8_e4m3fn


_GRP = 2


def _norm_kernel(x_ref, w_ref, wtb_ref, o_ref):
    w = w_ref[...]
    for g in range(_IPB // _GRP):
        imgs = range(g * _GRP, (g + 1) * _GRP)
        pieces = []
        for i in imgs:
            x = x_ref[i]
            pieces.append(x.astype(_F8))
            pieces.append((x * x).astype(_F8))
        h = jnp.concatenate(
            [_dot(p, w).astype(_F8) for p in pieces], axis=0
        )
        for b, s in enumerate(_SLAB_STARTS):
            rhs = jnp.concatenate(
                [h[k * _N + s : k * _N + s + 256] for k in range(2 * _GRP)], axis=1
            )
            vb = _dot(wtb_ref[b], rhs)
            for j, i in enumerate(imgs):
                mean = vb[:, 2 * j * _N : (2 * j + 1) * _N]
                meansq = vb[:, (2 * j + 1) * _N : (2 * j + 2) * _N]
                var = jnp.maximum(jnp.abs(meansq - mean * mean), 1e-20)
                xb = x_ref[i, 128 * b : 128 * (b + 1), :]
                z = (xb - mean) * jax.lax.rsqrt(var)
                o_ref[i, 128 * b : 128 * (b + 1), :] = jax.lax.clamp(
                    -_CLAMP, z, _CLAMP
                )


def kernel(x):
    b, c, h, wd = x.shape
    n_img = b * c
    xi = x.reshape(n_img, h, wd)
    w = jnp.asarray(_W_NP / 32.0, dtype=_F8)
    wtb = jnp.asarray(_WT_BLOCKS_NP / 32.0, dtype=_F8)
    out = pl.pallas_call(
        _norm_kernel,
        out_shape=jax.ShapeDtypeStruct((n_img, h, wd), x.dtype),
        grid=(n_img // _IPB,),
        in_specs=[
            pl.BlockSpec((_IPB, h, wd), lambda i: (i, 0, 0)),
            pl.BlockSpec((h, wd), lambda i: (0, 0)),
            pl.BlockSpec((4, 128, 256), lambda i: (0, 0, 0)),
        ],
        out_specs=pl.BlockSpec((_IPB, h, wd), lambda i: (i, 0, 0)),
        compiler_params=pltpu.CompilerParams(
            dimension_semantics=("arbitrary",),
            vmem_limit_bytes=56 * 1024 * 1024,
        ),
        name="local_norm2d",
    )(xi, w, wtb)
    return out.reshape(b, c, h, wd)
